# baseline (speedup 1.0000x reference)
import functools

import jax
import jax.numpy as jnp
from jax import lax
from jax.experimental import pallas as pl
from jax.experimental.pallas import tpu as pltpu
from jax.experimental.pallas import tpu_sc as plsc

_CHUNK = 32
_NBUF = 3


def kernel(x, emb):
    seq_len = x.shape[1]
    d = emb.shape[1]
    info = plsc.get_sparse_core_info()
    nc, ns = info.num_cores, info.num_subcores
    rows_w = seq_len // (nc * ns)
    nchunks = rows_w // _CHUNK
    mesh = plsc.VectorSubcoreMesh(core_axis_name="c", subcore_axis_name="s")

    @functools.partial(
        pl.kernel,
        out_type=jax.ShapeDtypeStruct((seq_len, d), emb.dtype),
        mesh=mesh,
        scratch_types=[
            pltpu.VMEM((_NBUF, _CHUNK, d), jnp.float32),
            pltpu.SemaphoreType.DMA((_NBUF,)),
            pltpu.SemaphoreType.DMA((_NBUF,)),
        ],
    )
    def run(emb_hbm, out_hbm, buf, rsems, wsems):
        wid = lax.axis_index("s") * nc + lax.axis_index("c")
        base = wid * rows_w

        def rd(i):
            return pltpu.make_async_copy(
                emb_hbm.at[pl.ds(base + i * _CHUNK, _CHUNK)],
                buf.at[i % _NBUF],
                rsems.at[i % _NBUF],
            )

        def wr(i):
            return pltpu.make_async_copy(
                buf.at[i % _NBUF],
                out_hbm.at[pl.ds(base + i * _CHUNK, _CHUNK)],
                wsems.at[i % _NBUF],
            )

        for i in range(_NBUF):
            rd(i).start()
        for i in range(nchunks):
            rd(i).wait()
            wr(i).start()
            if i + _NBUF < nchunks:
                wr(i).wait()
                rd(i + _NBUF).start()
        for i in range(nchunks - _NBUF, nchunks):
            wr(i).wait()

    return run(emb)
